# baseline (speedup 1.0000x reference)
import jax
import jax.numpy as jnp
from jax.experimental import pallas as pl
from jax.experimental.pallas import tpu as pltpu

NUM_LAYERS = 24
STM_SIZE = 2048
EMBED_DIM = 1024

_NSPLIT = 1
_CH = STM_SIZE // _NSPLIT


def _copy_body(layer_ref, mem_ref, out_ref, *bufs_and_sems):
    bufs = bufs_and_sems[:_NSPLIT]
    gsems, ssems = bufs_and_sems[_NSPLIT], bufs_and_sems[_NSPLIT + 1]
    l = layer_ref[0]
    gathers = [
        pltpu.make_async_copy(
            mem_ref.at[l, pl.ds(i * _CH, _CH)], bufs[i], gsems.at[i])
        for i in range(_NSPLIT)
    ]
    scatters = [
        pltpu.make_async_copy(
            bufs[i], out_ref.at[pl.ds(i * _CH, _CH)], ssems.at[i])
        for i in range(_NSPLIT)
    ]
    for c in gathers:
        c.start()
    for i in range(_NSPLIT):
        gathers[i].wait()
        scatters[i].start()
    for c in scatters:
        c.wait()


@jax.jit
def _tc_copy_layer(mem3, layer_arr):
    return pl.pallas_call(
        _copy_body,
        in_specs=[
            pl.BlockSpec(memory_space=pltpu.SMEM),
            pl.BlockSpec(memory_space=pl.ANY),
        ],
        out_specs=pl.BlockSpec(memory_space=pl.ANY),
        out_shape=jax.ShapeDtypeStruct((STM_SIZE, EMBED_DIM), jnp.float32),
        scratch_shapes=(
            [pltpu.VMEM((_CH, EMBED_DIM), jnp.float32) for _ in range(_NSPLIT)]
            + [pltpu.SemaphoreType.DMA((_NSPLIT,)),
               pltpu.SemaphoreType.DMA((_NSPLIT,))]
        ),
    )(layer_arr, mem3)


def kernel(memory, layer):
    mem3 = memory.reshape(NUM_LAYERS, STM_SIZE, EMBED_DIM)
    layer_arr = jnp.asarray(layer, jnp.int32).reshape(1)
    out = _tc_copy_layer(mem3, layer_arr)
    return out.reshape(1, STM_SIZE, EMBED_DIM)


# speedup vs baseline: 1.1058x; 1.1058x over previous
import jax
import jax.numpy as jnp
from jax.experimental import pallas as pl
from jax.experimental.pallas import tpu as pltpu

NUM_LAYERS = 24
STM_SIZE = 2048
EMBED_DIM = 1024

_CHUNKS = [1024, 512, 256, 256]
_OFFS = [0, 1024, 1536, 1792]
_NSPLIT = len(_CHUNKS)


def _copy_body(layer_ref, mem_ref, out_ref, *bufs_and_sems):
    bufs = bufs_and_sems[:_NSPLIT]
    gsems, ssems = bufs_and_sems[_NSPLIT], bufs_and_sems[_NSPLIT + 1]
    l = layer_ref[0]
    gathers = [
        pltpu.make_async_copy(
            mem_ref.at[l, pl.ds(_OFFS[i], _CHUNKS[i])], bufs[i], gsems.at[i])
        for i in range(_NSPLIT)
    ]
    scatters = [
        pltpu.make_async_copy(
            bufs[i], out_ref.at[pl.ds(_OFFS[i], _CHUNKS[i])], ssems.at[i])
        for i in range(_NSPLIT)
    ]
    for c in gathers:
        c.start()
    for i in range(_NSPLIT):
        gathers[i].wait()
        scatters[i].start()
    for c in scatters:
        c.wait()


@jax.jit
def _tc_copy_layer(mem3, layer_arr):
    return pl.pallas_call(
        _copy_body,
        in_specs=[
            pl.BlockSpec(memory_space=pltpu.SMEM),
            pl.BlockSpec(memory_space=pl.ANY),
        ],
        out_specs=pl.BlockSpec(memory_space=pl.ANY),
        out_shape=jax.ShapeDtypeStruct((STM_SIZE, EMBED_DIM), jnp.float32),
        scratch_shapes=(
            [pltpu.VMEM((c, EMBED_DIM), jnp.float32) for c in _CHUNKS]
            + [pltpu.SemaphoreType.DMA((_NSPLIT,)),
               pltpu.SemaphoreType.DMA((_NSPLIT,))]
        ),
    )(layer_arr, mem3)


def kernel(memory, layer):
    mem3 = memory.reshape(NUM_LAYERS, STM_SIZE, EMBED_DIM)
    layer_arr = jnp.asarray(layer, jnp.int32).reshape(1)
    out = _tc_copy_layer(mem3, layer_arr)
    return out.reshape(1, STM_SIZE, EMBED_DIM)


# speedup vs baseline: 1.1109x; 1.0046x over previous
import jax
import jax.numpy as jnp
from jax.experimental import pallas as pl
from jax.experimental.pallas import tpu as pltpu

NUM_LAYERS = 24
STM_SIZE = 2048
EMBED_DIM = 1024

_CHUNKS = [1024, 768, 256]
_OFFS = [0, 1024, 1792]
_NSPLIT = len(_CHUNKS)


def _copy_body(layer_ref, mem_ref, out_ref, *bufs_and_sems):
    bufs = bufs_and_sems[:_NSPLIT]
    gsems, ssems = bufs_and_sems[_NSPLIT], bufs_and_sems[_NSPLIT + 1]
    l = layer_ref[0]
    gathers = [
        pltpu.make_async_copy(
            mem_ref.at[l, pl.ds(_OFFS[i], _CHUNKS[i])], bufs[i], gsems.at[i])
        for i in range(_NSPLIT)
    ]
    scatters = [
        pltpu.make_async_copy(
            bufs[i], out_ref.at[pl.ds(_OFFS[i], _CHUNKS[i])], ssems.at[i])
        for i in range(_NSPLIT)
    ]
    for c in gathers:
        c.start()
    for i in range(_NSPLIT):
        gathers[i].wait()
        scatters[i].start()
    for c in scatters:
        c.wait()


@jax.jit
def _tc_copy_layer(mem3, layer_arr):
    return pl.pallas_call(
        _copy_body,
        in_specs=[
            pl.BlockSpec(memory_space=pltpu.SMEM),
            pl.BlockSpec(memory_space=pl.ANY),
        ],
        out_specs=pl.BlockSpec(memory_space=pl.ANY),
        out_shape=jax.ShapeDtypeStruct((STM_SIZE, EMBED_DIM), jnp.float32),
        scratch_shapes=(
            [pltpu.VMEM((c, EMBED_DIM), jnp.float32) for c in _CHUNKS]
            + [pltpu.SemaphoreType.DMA((_NSPLIT,)),
               pltpu.SemaphoreType.DMA((_NSPLIT,))]
        ),
    )(layer_arr, mem3)


def kernel(memory, layer):
    mem3 = memory.reshape(NUM_LAYERS, STM_SIZE, EMBED_DIM)
    layer_arr = jnp.asarray(layer, jnp.int32).reshape(1)
    out = _tc_copy_layer(mem3, layer_arr)
    return out.reshape(1, STM_SIZE, EMBED_DIM)
